# baseline (speedup 1.0000x reference)
import dataclasses

import jax
import jax.numpy as jnp
from jax.experimental import pallas as pl
from jax.experimental.pallas import tpu as pltpu
from jax.experimental.pallas import tpu_sc as plsc

_SC_COMPILER_PARAMS = pltpu.CompilerParams()
if "needs_layout_passes" in pltpu.CompilerParams.__dataclass_fields__:
    _SC_COMPILER_PARAMS = dataclasses.replace(
        _SC_COMPILER_PARAMS, needs_layout_passes=False)

_B, _N, _C = 8, 4096, 16
_K = _N // 2
_NUM_WORKERS = 32
_WPB = _NUM_WORKERS // _B
_ROWS = _K // _WPB


_CHUNK = 128
_NPART = _N // _CHUNK


def _tree(parts, op):
    parts = list(parts)
    while len(parts) > 1:
        nxt = [op(parts[i], parts[i + 1]) for i in range(0, len(parts) - 1, 2)]
        if len(parts) % 2:
            nxt.append(parts[-1])
        parts = nxt
    return parts[0]


def _lane_fold(v, op):
    for s in (64, 32, 16, 8, 4, 2, 1):
        v = op(v, pltpu.roll(v, s, 1))
    return v


def _fps_body(x_ref, y_ref, z_ref, idx_ref):
    xs = [x_ref[:, k * _CHUNK:(k + 1) * _CHUNK] for k in range(_NPART)]
    ys = [y_ref[:, k * _CHUNK:(k + 1) * _CHUNK] for k in range(_NPART)]
    zs = [z_ref[:, k * _CHUNK:(k + 1) * _CHUNK] for k in range(_NPART)]
    lane = jax.lax.broadcasted_iota(jnp.int32, (_B, _CHUNK), 1)
    iotas = [lane + jnp.int32(k * _CHUNK) for k in range(_NPART)]

    neg_inf = jnp.float32(-jnp.inf)

    min_d0 = tuple(jnp.full((_B, _CHUNK), jnp.inf, jnp.float32)
                   for _ in range(_NPART))

    def inner(j, state):
        min_d, buf = state
        m = _lane_fold(_tree(min_d, jnp.maximum), jnp.maximum)
        cand = [jnp.where(min_d[k] == m, iotas[k], jnp.int32(_N))
                for k in range(_NPART)]
        sel = _lane_fold(_tree(cand, jnp.minimum), jnp.minimum)
        oh = [iotas[k] == sel for k in range(_NPART)]
        px = _lane_fold(
            _tree([jnp.where(oh[k], xs[k], neg_inf) for k in range(_NPART)],
                  jnp.maximum), jnp.maximum)
        py = _lane_fold(
            _tree([jnp.where(oh[k], ys[k], neg_inf) for k in range(_NPART)],
                  jnp.maximum), jnp.maximum)
        pz = _lane_fold(
            _tree([jnp.where(oh[k], zs[k], neg_inf) for k in range(_NPART)],
                  jnp.maximum), jnp.maximum)

        def upd(k):
            dx = xs[k] - px
            dy = ys[k] - py
            dz = zs[k] - pz
            d = (dx * dx + dz * dz) + dy * dy
            return jnp.minimum(min_d[k], d)

        new_min_d = tuple(upd(k) for k in range(_NPART))
        buf = jnp.where(lane == j, sel, buf)
        return new_min_d, buf

    def outer(c, min_d):
        buf = jnp.zeros((_B, _CHUNK), jnp.int32)
        min_d, buf = jax.lax.fori_loop(0, _CHUNK, inner, (min_d, buf))
        idx_ref[:, pl.ds(pl.multiple_of(c * _CHUNK, _CHUNK), _CHUNK)] = buf
        return min_d

    jax.lax.fori_loop(0, _K // _CHUNK, outer, min_d0)


def _fps_indices(x, y, z):
    return pl.pallas_call(
        _fps_body,
        out_shape=jax.ShapeDtypeStruct((_B, _K), jnp.int32),
    )(x, y, z)


def _sc_gather(points_flat, idx):
    @pl.kernel(
        out_type=jax.ShapeDtypeStruct((_B, _K * _C), points_flat.dtype),
        mesh=plsc.VectorSubcoreMesh(core_axis_name="core",
                                    subcore_axis_name="subcore"),
        compiler_params=_SC_COMPILER_PARAMS,
        scratch_types=[
            pltpu.VMEM((_N * _C,), jnp.float32),
            pltpu.VMEM((_ROWS,), jnp.int32),
            pltpu.VMEM((_ROWS * _C,), jnp.float32),
            pltpu.SemaphoreType.DMA,
        ],
    )
    def gather_kernel(x_hbm, i_hbm, o_hbm, pts_v, idx_v, out_v, sem):
        core = jax.lax.axis_index("core")
        sub = jax.lax.axis_index("subcore")
        wid = core * 16 + sub
        batch = wid // _WPB
        slot = wid % _WPB

        pltpu.async_copy(x_hbm.at[batch], pts_v, sem).wait()
        pltpu.async_copy(i_hbm.at[batch, pl.ds(slot * _ROWS, _ROWS)],
                         idx_v, sem).wait()

        lane = jax.lax.iota(jnp.int32, 16)

        @pl.loop(0, _ROWS)
        def _(j):
            bidx = plsc.load_gather(idx_v, [jnp.full((16,), j, jnp.int32)])
            row = plsc.load_gather(pts_v, [bidx * _C + lane])
            out_v[pl.ds(j * _C, _C)] = row

        pltpu.async_copy(out_v,
                         o_hbm.at[batch, pl.ds(slot * _ROWS * _C, _ROWS * _C)],
                         sem).wait()

    return gather_kernel(points_flat, idx)


def kernel(xyz):
    B, N, C = xyz.shape
    x = xyz[:, :, 0]
    y = xyz[:, :, 1]
    z = xyz[:, :, 2]
    idx = _fps_indices(x, y, z)
    gathered = _sc_gather(xyz.reshape(B, N * C), idx)
    return gathered.reshape(B, _K, C)


# speedup vs baseline: 4.1501x; 4.1501x over previous
import dataclasses

import jax
import jax.numpy as jnp
from jax.experimental import pallas as pl
from jax.experimental.pallas import tpu as pltpu
from jax.experimental.pallas import tpu_sc as plsc

_SC_COMPILER_PARAMS = pltpu.CompilerParams()
if "needs_layout_passes" in pltpu.CompilerParams.__dataclass_fields__:
    _SC_COMPILER_PARAMS = dataclasses.replace(
        _SC_COMPILER_PARAMS, needs_layout_passes=False)

_B, _N, _C = 8, 4096, 16
_K = _N // 2
_NUM_WORKERS = 32
_WPB = _NUM_WORKERS // _B
_ROWS = _K // _WPB


_CHUNK = 128


def _fps_body(x_ref, y_ref, z_ref, idx_ref):
    x = x_ref[...]
    y = y_ref[...]
    z = z_ref[...]
    iota_f = jax.lax.broadcasted_iota(jnp.int32, (_B, _N), 1).astype(jnp.float32)
    lane = jax.lax.broadcasted_iota(jnp.int32, (_B, _CHUNK), 1)

    neg_inf = jnp.float32(-jnp.inf)
    pos_inf = jnp.float32(jnp.inf)

    min_d0 = jnp.full((_B, _N), jnp.inf, jnp.float32)

    def inner(j, state):
        min_d, buf = state
        m = jnp.max(min_d, axis=1, keepdims=True)
        cand = jnp.where(min_d == m, iota_f, pos_inf)
        sel = jnp.min(cand, axis=1, keepdims=True)
        onehot = iota_f == sel
        px = jnp.max(jnp.where(onehot, x, neg_inf), axis=1, keepdims=True)
        py = jnp.max(jnp.where(onehot, y, neg_inf), axis=1, keepdims=True)
        pz = jnp.max(jnp.where(onehot, z, neg_inf), axis=1, keepdims=True)
        dx = x - px
        dy = y - py
        dz = z - pz
        d = (dx * dx + dz * dz) + dy * dy
        buf = jnp.where(lane == j, sel.astype(jnp.int32), buf)
        return jnp.minimum(min_d, d), buf

    def outer(c, min_d):
        buf = jnp.zeros((_B, _CHUNK), jnp.int32)
        min_d, buf = jax.lax.fori_loop(0, _CHUNK, inner, (min_d, buf))
        idx_ref[:, pl.ds(pl.multiple_of(c * _CHUNK, _CHUNK), _CHUNK)] = buf
        return min_d

    jax.lax.fori_loop(0, _K // _CHUNK, outer, min_d0)


def _fps_indices(x, y, z):
    return pl.pallas_call(
        _fps_body,
        out_shape=jax.ShapeDtypeStruct((_B, _K), jnp.int32),
    )(x, y, z)


def _sc_gather(points_flat, idx):
    @pl.kernel(
        out_type=jax.ShapeDtypeStruct((_B, _K * _C), points_flat.dtype),
        mesh=plsc.VectorSubcoreMesh(core_axis_name="core",
                                    subcore_axis_name="subcore"),
        compiler_params=_SC_COMPILER_PARAMS,
        scratch_types=[
            pltpu.VMEM((_N * _C,), jnp.float32),
            pltpu.VMEM((_ROWS,), jnp.int32),
            pltpu.VMEM((_ROWS * _C,), jnp.float32),
            pltpu.SemaphoreType.DMA,
        ],
    )
    def gather_kernel(x_hbm, i_hbm, o_hbm, pts_v, idx_v, out_v, sem):
        core = jax.lax.axis_index("core")
        sub = jax.lax.axis_index("subcore")
        wid = core * 16 + sub
        batch = wid // _WPB
        slot = wid % _WPB

        pltpu.async_copy(x_hbm.at[batch], pts_v, sem).wait()
        pltpu.async_copy(i_hbm.at[batch, pl.ds(slot * _ROWS, _ROWS)],
                         idx_v, sem).wait()

        lane = jax.lax.iota(jnp.int32, 16)

        @pl.loop(0, _ROWS)
        def _(j):
            bidx = plsc.load_gather(idx_v, [jnp.full((16,), j, jnp.int32)])
            row = plsc.load_gather(pts_v, [bidx * _C + lane])
            out_v[pl.ds(j * _C, _C)] = row

        pltpu.async_copy(out_v,
                         o_hbm.at[batch, pl.ds(slot * _ROWS * _C, _ROWS * _C)],
                         sem).wait()

    return gather_kernel(points_flat, idx)


def kernel(xyz):
    B, N, C = xyz.shape
    x = xyz[:, :, 0]
    y = xyz[:, :, 1]
    z = xyz[:, :, 2]
    idx = _fps_indices(x, y, z)
    gathered = _sc_gather(xyz.reshape(B, N * C), idx)
    return gathered.reshape(B, _K, C)


# speedup vs baseline: 4.3403x; 1.0458x over previous
import dataclasses

import jax
import jax.numpy as jnp
from jax.experimental import pallas as pl
from jax.experimental.pallas import tpu as pltpu
from jax.experimental.pallas import tpu_sc as plsc

_SC_COMPILER_PARAMS = pltpu.CompilerParams()
if "needs_layout_passes" in pltpu.CompilerParams.__dataclass_fields__:
    _SC_COMPILER_PARAMS = dataclasses.replace(
        _SC_COMPILER_PARAMS, needs_layout_passes=False)

_B, _N, _C = 8, 4096, 16
_K = _N // 2
_NUM_WORKERS = 32
_WPB = _NUM_WORKERS // _B
_ROWS = _K // _WPB


_CHUNK = 128


def _fps_body(x_ref, y_ref, z_ref, idx_ref):
    x = x_ref[...]
    y = y_ref[...]
    z = z_ref[...]
    iota_f = jax.lax.broadcasted_iota(jnp.int32, (_B, _N), 1).astype(jnp.float32)
    lane = jax.lax.broadcasted_iota(jnp.int32, (_B, _CHUNK), 1)

    neg_inf = jnp.float32(-jnp.inf)
    pos_inf = jnp.float32(jnp.inf)

    min_d0 = jnp.full((_B, _N), jnp.inf, jnp.float32)

    def inner(j, state):
        min_d, buf = state
        m = jnp.max(min_d, axis=1, keepdims=True)
        cand = jnp.where(min_d == m, iota_f, pos_inf)
        sel = jnp.min(cand, axis=1, keepdims=True)
        onehot = iota_f == sel
        px = jnp.max(jnp.where(onehot, x, neg_inf), axis=1, keepdims=True)
        py = jnp.max(jnp.where(onehot, y, neg_inf), axis=1, keepdims=True)
        pz = jnp.max(jnp.where(onehot, z, neg_inf), axis=1, keepdims=True)
        dx = x - px
        dy = y - py
        dz = z - pz
        d = (dx * dx + dz * dz) + dy * dy
        buf = jnp.where(lane == j, sel.astype(jnp.int32), buf)
        return jnp.minimum(min_d, d), buf

    def outer(c, min_d):
        buf = jnp.zeros((_B, _CHUNK), jnp.int32)
        min_d, buf = jax.lax.fori_loop(0, _CHUNK, inner, (min_d, buf),
                                       unroll=4)
        idx_ref[:, pl.ds(pl.multiple_of(c * _CHUNK, _CHUNK), _CHUNK)] = buf
        return min_d

    jax.lax.fori_loop(0, _K // _CHUNK, outer, min_d0)


def _fps_indices(x, y, z):
    return pl.pallas_call(
        _fps_body,
        out_shape=jax.ShapeDtypeStruct((_B, _K), jnp.int32),
    )(x, y, z)


def _sc_gather(points_flat, idx):
    @pl.kernel(
        out_type=jax.ShapeDtypeStruct((_B, _K * _C), points_flat.dtype),
        mesh=plsc.VectorSubcoreMesh(core_axis_name="core",
                                    subcore_axis_name="subcore"),
        compiler_params=_SC_COMPILER_PARAMS,
        scratch_types=[
            pltpu.VMEM((_N * _C,), jnp.float32),
            pltpu.VMEM((_ROWS,), jnp.int32),
            pltpu.VMEM((_ROWS * _C,), jnp.float32),
            pltpu.SemaphoreType.DMA,
        ],
    )
    def gather_kernel(x_hbm, i_hbm, o_hbm, pts_v, idx_v, out_v, sem):
        core = jax.lax.axis_index("core")
        sub = jax.lax.axis_index("subcore")
        wid = core * 16 + sub
        batch = wid // _WPB
        slot = wid % _WPB

        pltpu.async_copy(x_hbm.at[batch], pts_v, sem).wait()
        pltpu.async_copy(i_hbm.at[batch, pl.ds(slot * _ROWS, _ROWS)],
                         idx_v, sem).wait()

        lane = jax.lax.iota(jnp.int32, 16)

        @pl.loop(0, _ROWS)
        def _(j):
            bidx = plsc.load_gather(idx_v, [jnp.full((16,), j, jnp.int32)])
            row = plsc.load_gather(pts_v, [bidx * _C + lane])
            out_v[pl.ds(j * _C, _C)] = row

        pltpu.async_copy(out_v,
                         o_hbm.at[batch, pl.ds(slot * _ROWS * _C, _ROWS * _C)],
                         sem).wait()

    return gather_kernel(points_flat, idx)


def kernel(xyz):
    B, N, C = xyz.shape
    x = xyz[:, :, 0]
    y = xyz[:, :, 1]
    z = xyz[:, :, 2]
    idx = _fps_indices(x, y, z)
    gathered = _sc_gather(xyz.reshape(B, N * C), idx)
    return gathered.reshape(B, _K, C)
